# Initial kernel scaffold; baseline (speedup 1.0000x reference)
import jax
import jax.numpy as jnp
from jax.experimental import pallas as pl


def kernel(Z, Y):
    raise NotImplementedError("write your pallas kernel here")


# speedup vs baseline: 6.2630x; 6.2630x over previous
import jax
import jax.numpy as jnp
from jax.experimental import pallas as pl
from jax.experimental.pallas import tpu as pltpu

_BI = 1024
_BJ = 1024


def _sim_kernel(y_ref, z_ref, sim_ref, gt_ref, eq_ref, d_ref):
    i = pl.program_id(0)
    j = pl.program_id(1)
    nj = pl.num_programs(1)
    bi, bj = sim_ref.shape

    yb = y_ref[...]
    zb = z_ref[...]
    yn = yb * (1.0 / jnp.sqrt(jnp.sum(yb * yb, axis=1, keepdims=True)))
    zn = zb * (1.0 / jnp.sqrt(jnp.sum(zb * zb, axis=1, keepdims=True)))
    s = jax.lax.dot_general(
        yn, zn, (((1,), (1,)), ((), ())), preferred_element_type=jnp.float32)
    sim_ref[...] = s

    j_actual = jax.lax.rem(i + j, nj)
    row_g = i * bi + jax.lax.broadcasted_iota(jnp.int32, (bi, bj), 0)
    col_g = j_actual * bj + jax.lax.broadcasted_iota(jnp.int32, (bi, bj), 1)

    @pl.when(j == 0)
    def _():
        d_ref[...] = jnp.sum(
            jnp.where(row_g == col_g, s, 0.0), axis=1, keepdims=True)

    d = d_ref[...]
    gt = jnp.sum((s > d).astype(jnp.float32), axis=1, keepdims=True)
    eq = jnp.sum(((s == d) & (col_g < row_g)).astype(jnp.float32),
                 axis=1, keepdims=True)

    @pl.when(j == 0)
    def _():
        gt_ref[...] = gt
        eq_ref[...] = eq

    @pl.when(j != 0)
    def _():
        gt_ref[...] += gt
        eq_ref[...] += eq


def _acc_kernel(gt_ref, eq_ref, t1_ref, t10_ref):
    gt = gt_ref[...]
    eq = eq_ref[...]
    n = gt.shape[0]
    t1_ref[0, 0] = jnp.sum(
        ((gt == 0.0) & (eq == 0.0)).astype(jnp.float32)) * (1.0 / n)
    t10_ref[0, 0] = jnp.sum(
        (gt + eq < 10.0).astype(jnp.float32)) * (1.0 / n)


def kernel(Z, Y):
    b, f = Z.shape
    ni = b // _BI
    nj = b // _BJ

    sim, gt, eq = pl.pallas_call(
        _sim_kernel,
        grid=(ni, nj),
        in_specs=[
            pl.BlockSpec((_BI, f), lambda i, j: (i, 0)),
            pl.BlockSpec((_BJ, f), lambda i, j: ((i + j) % nj, 0)),
        ],
        out_specs=[
            pl.BlockSpec((_BI, _BJ), lambda i, j: (i, (i + j) % nj)),
            pl.BlockSpec((_BI, 1), lambda i, j: (i, 0)),
            pl.BlockSpec((_BI, 1), lambda i, j: (i, 0)),
        ],
        out_shape=[
            jax.ShapeDtypeStruct((b, b), jnp.float32),
            jax.ShapeDtypeStruct((b, 1), jnp.float32),
            jax.ShapeDtypeStruct((b, 1), jnp.float32),
        ],
        scratch_shapes=[pltpu.VMEM((_BI, 1), jnp.float32)],
        compiler_params=pltpu.CompilerParams(
            dimension_semantics=("parallel", "arbitrary"),
        ),
    )(Y, Z)

    t1, t10 = pl.pallas_call(
        _acc_kernel,
        out_specs=[
            pl.BlockSpec(memory_space=pltpu.SMEM),
            pl.BlockSpec(memory_space=pltpu.SMEM),
        ],
        out_shape=[
            jax.ShapeDtypeStruct((1, 1), jnp.float32),
            jax.ShapeDtypeStruct((1, 1), jnp.float32),
        ],
    )(gt, eq)

    return (t1[0, 0], t10[0, 0], sim)
